# baseline (speedup 1.0000x reference)
import jax
import jax.numpy as jnp
from jax import lax
from jax.experimental import pallas as pl
from jax.experimental.pallas import tpu as pltpu
from jax.experimental.pallas import tpu_sc as plsc

B = 16384
N2 = 361
PADN = 368
NGRP = PADN // 16
TBL = 512
NBKT = 8192
BSH = 18
NW = 32
GPT = B // NW
CH = 32
NCHUNK = GPT // CH
I32MAX = 2147483647


def _sc_body(hist_hbm, legal_hbm, sd_hbm, rank_hbm, off_hbm, cur_hbm, pl_hbm,
             mc_hbm, out_hbm, sd_v, rank_v, off_v, cur_v, pl_v, mc_v,
             present_v, hist0, hist1, legal0, legal1, out_buf, sem0, sem1):
    wid = lax.axis_index("s") * 2 + lax.axis_index("c")
    base = wid * GPT

    pltpu.sync_copy(sd_hbm, sd_v)
    pltpu.sync_copy(rank_hbm, rank_v)
    pltpu.sync_copy(off_hbm, off_v)
    pltpu.sync_copy(cur_hbm.at[pl.ds(base, GPT)], cur_v.at[pl.ds(0, GPT)])
    pltpu.sync_copy(pl_hbm.at[pl.ds(base, GPT)], pl_v.at[pl.ds(0, GPT)])
    pltpu.sync_copy(mc_hbm.at[pl.ds(base, GPT)], mc_v.at[pl.ds(0, GPT)])

    lanes = lax.iota(jnp.int32, 16)
    for k in range(TBL // 16):
        present_v[pl.ds(k * 16, 16)] = jnp.full((16,), -1, jnp.int32)

    bufs = ((hist0, legal0, sem0), (hist1, legal1, sem1))

    def in_copies(cc, hist_s, legal_s, sem):
        rows = pl.ds(base + cc * CH, CH)
        return (pltpu.make_async_copy(hist_hbm.at[rows], hist_s, sem),
                pltpu.make_async_copy(legal_hbm.at[rows], legal_s, sem))

    for s in range(2):
        for cp in in_copies(s, *bufs[s]):
            cp.start()

    def chunk_compute(cc, hist_s, legal_s):
        def game_body(gi, _):
            g = cc * CH + gi
            cur = cur_v[pl.ds(g, 16)][0]
            player = lax.bitwise_and(pl_v[pl.ds(g, 16)][0], 1)
            L = lax.min(mc_v[pl.ds(g, 16)][0], N2)
            gvec = jnp.full((16,), g, jnp.int32)
            sd_base = jnp.full((16,), player * TBL, jnp.int32)
            off_base = jnp.full((16,), player * NBKT, jnp.int32)
            ngrp = (L + 15) // 16
            zero16 = jnp.zeros((16,), jnp.int32)

            @plsc.parallel_loop(0, ngrp, unroll=4, carry=zero16)
            def probe_fast(jg, acc):
                jbase = jnp.minimum(jg * 16, N2 - 16)
                valid = (jbase + lanes) < L
                t = hist_s[gi, pl.ds(jbase, 16)] ^ cur
                r = lax.shift_right_logical(t, BSH) & (NBKT - 1)
                pos = plsc.load_gather(off_v, [off_base + r])
                v = plsc.load_gather(sd_v, [sd_base + pos])
                pos = pos + jnp.where(v < t, 1, 0).astype(jnp.int32)
                v = plsc.load_gather(sd_v, [sd_base + pos])
                pos = pos + jnp.where(v < t, 1, 0).astype(jnp.int32)
                v = plsc.load_gather(sd_v, [sd_base + pos])
                found = (v == t) & valid
                plsc.store_scatter(present_v, [pos], gvec, mask=found)
                acc = acc | jnp.where(valid & (v < t), 1, 0).astype(jnp.int32)
                return acc | jnp.where(found, 2, 0).astype(jnp.int32)

            acc_m = jnp.max(probe_fast)
            need_slow = (acc_m & 1) > 0

            def probe_slow(jg, _):
                jbase = jnp.minimum(jg * 16, N2 - 16)
                valid = (jbase + lanes) < L
                t = hist_s[gi, pl.ds(jbase, 16)] ^ cur
                r = lax.shift_right_logical(t, BSH) & (NBKT - 1)
                pos = plsc.load_gather(off_v, [off_base + r])
                v = plsc.load_gather(sd_v, [sd_base + pos])

                def scan_cond(c):
                    _, vv = c
                    return jnp.any(vv < t)

                def scan_step(c):
                    pp, vv = c
                    pp = pp + jnp.where(vv < t, 1, 0).astype(jnp.int32)
                    return pp, plsc.load_gather(sd_v, [sd_base + pp])

                pos, v = lax.while_loop(scan_cond, scan_step, (pos, v))
                found = (v == t) & valid
                plsc.store_scatter(present_v, [pos], gvec, mask=found)
                return 0

            @pl.when(need_slow)
            def _():
                lax.fori_loop(0, ngrp, probe_slow, 0)

            game_hit = (acc_m >= 2) | need_slow
            rk0 = player * PADN

            @pl.when(game_hit)
            def _():
                for pg in range(NGRP):
                    pb = min(pg * 16, N2 - 16)
                    r = rank_v[pl.ds(rk0 + pb, 16)]
                    rep = plsc.load_gather(present_v, [r]) == gvec
                    lg = legal_s[gi, pl.ds(pb, 16)]
                    out_buf[gi, pl.ds(pb, 16)] = jnp.where(
                        (lg > 0) & rep, jnp.float32(0), lg)

            @pl.when(jnp.logical_not(game_hit))
            def _():
                for pg in range(NGRP):
                    pb = min(pg * 16, N2 - 16)
                    out_buf[gi, pl.ds(pb, 16)] = legal_s[gi, pl.ds(pb, 16)]
            return 0

        lax.fori_loop(0, CH, game_body, 0)

    def pair_body(cpair, _):
        for s in range(2):
            cc = cpair * 2 + s
            hist_s, legal_s, sem = bufs[s]
            for cp in in_copies(cc, hist_s, legal_s, sem):
                cp.wait()
            chunk_compute(cc, hist_s, legal_s)

            @pl.when(cc + 2 < NCHUNK)
            def _():
                for cp in in_copies(cc + 2, hist_s, legal_s, sem):
                    cp.start()

            pltpu.sync_copy(out_buf, out_hbm.at[pl.ds(base + cc * CH, CH)])
        return 0

    lax.fori_loop(0, NCHUNK // 2, pair_body, 0)


def kernel(legal_mask, Zpos, current_player, current_hash, hash_history,
           move_count):
    b, h, w = legal_mask.shape
    d = Zpos[:, 0][None, :] ^ jnp.stack([Zpos[:, 1], Zpos[:, 2]])
    sd = jnp.sort(d, axis=1)
    sd_pad = jnp.concatenate(
        [sd, jnp.full((2, TBL - N2), I32MAX, jnp.int32)], axis=1)
    rank = jnp.sum(d[:, None, :] < d[:, :, None], axis=2,
                   dtype=jnp.int32)
    rank_pad = jnp.concatenate(
        [rank, jnp.zeros((2, PADN - N2), jnp.int32)], axis=1)
    buck = lax.shift_right_logical(d, BSH)
    off = jnp.sum(
        buck[:, None, :] < jnp.arange(NBKT, dtype=jnp.int32)[None, :, None],
        axis=2, dtype=jnp.int32)


    mesh = plsc.VectorSubcoreMesh(core_axis_name="c", subcore_axis_name="s")
    kfn = pl.kernel(
        _sc_body,
        mesh=mesh,
        out_type=jax.ShapeDtypeStruct((B, N2), jnp.float32),
        compiler_params=pltpu.CompilerParams(needs_layout_passes=False,
                                             use_tc_tiling_on_sc=False),
        scratch_types=[
            pltpu.VMEM((2 * TBL,), jnp.int32),
            pltpu.VMEM((2 * PADN,), jnp.int32),
            pltpu.VMEM((2 * NBKT,), jnp.int32),
            pltpu.VMEM((GPT + 16,), jnp.int32),
            pltpu.VMEM((GPT + 16,), jnp.int32),
            pltpu.VMEM((GPT + 16,), jnp.int32),
            pltpu.VMEM((TBL,), jnp.int32),
            pltpu.VMEM((CH, N2), jnp.int32),
            pltpu.VMEM((CH, N2), jnp.int32),
            pltpu.VMEM((CH, N2), jnp.float32),
            pltpu.VMEM((CH, N2), jnp.float32),
            pltpu.VMEM((CH, N2), jnp.float32),
            pltpu.SemaphoreType.DMA,
            pltpu.SemaphoreType.DMA,
        ],
    )
    out = kfn(hash_history, legal_mask.reshape(b, N2), sd_pad.reshape(-1),
              rank_pad.reshape(-1), off.reshape(-1), current_hash,
              current_player, move_count)
    return out.reshape(b, h, w)


# speedup vs baseline: 1.3185x; 1.3185x over previous
import jax
import jax.numpy as jnp
from jax import lax
from jax.experimental import pallas as pl
from jax.experimental.pallas import tpu as pltpu
from jax.experimental.pallas import tpu_sc as plsc

B = 16384
N2 = 361
PADN = 368
NGRP = PADN // 16
TBL = 512
NBKT = 8192
BSH = 18
NW = 32
GPT = B // NW
CH = 32
NCHUNK = GPT // CH
I32MAX = 2147483647


def _sc_body(hist_hbm, legal_hbm, sd_hbm, rank_hbm, off_hbm, cur_hbm, pl_hbm,
             mc_hbm, out_hbm, sd_v, rank_v, off_v, cur_v, pl_v, mc_v,
             present_v, hist0, hist1, legal0, legal1, out_buf, sem0, sem1):
    wid = lax.axis_index("s") * 2 + lax.axis_index("c")
    base = wid * GPT

    pltpu.sync_copy(sd_hbm, sd_v)
    pltpu.sync_copy(rank_hbm, rank_v)
    pltpu.sync_copy(off_hbm, off_v)
    pltpu.sync_copy(cur_hbm.at[pl.ds(base, GPT)], cur_v.at[pl.ds(0, GPT)])
    pltpu.sync_copy(pl_hbm.at[pl.ds(base, GPT)], pl_v.at[pl.ds(0, GPT)])
    pltpu.sync_copy(mc_hbm.at[pl.ds(base, GPT)], mc_v.at[pl.ds(0, GPT)])

    lanes = lax.iota(jnp.int32, 16)
    for k in range(TBL // 16):
        present_v[pl.ds(k * 16, 16)] = jnp.full((16,), -1, jnp.int32)

    bufs = ((hist0, legal0, sem0), (hist1, legal1, sem1))

    def in_copies(cc, hist_s, legal_s, sem):
        rows = pl.ds(base + cc * CH, CH)
        return (pltpu.make_async_copy(hist_hbm.at[rows], hist_s, sem),
                pltpu.make_async_copy(legal_hbm.at[rows], legal_s, sem))

    for s in range(2):
        for cp in in_copies(s, *bufs[s]):
            cp.start()

    def chunk_compute(cc, hist_s, legal_s):
        def game_body(gi, _):
            g = cc * CH + gi
            cur = cur_v[pl.ds(g, 16)][0]
            player = lax.bitwise_and(pl_v[pl.ds(g, 16)][0], 1)
            L = lax.min(mc_v[pl.ds(g, 16)][0], N2)
            gvec = jnp.full((16,), g, jnp.int32)
            sd_base = jnp.full((16,), player * TBL, jnp.int32)
            off_base = jnp.full((16,), player * NBKT, jnp.int32)
            ngrp = (L + 15) // 16
            zero16 = jnp.zeros((16,), jnp.int32)

            @plsc.parallel_loop(0, ngrp, unroll=4, carry=zero16)
            def probe_fast(jg, acc):
                jbase = jg * 16
                valid = (jbase + lanes) < L
                t = hist_s[gi, pl.ds(jbase, 16)] ^ cur
                r = lax.shift_right_logical(t, BSH) & (NBKT - 1)
                pos = plsc.load_gather(off_v, [off_base + r])
                v = plsc.load_gather(sd_v, [sd_base + pos])
                pos = pos + jnp.where(v < t, 1, 0).astype(jnp.int32)
                v = plsc.load_gather(sd_v, [sd_base + pos])
                pos = pos + jnp.where(v < t, 1, 0).astype(jnp.int32)
                v = plsc.load_gather(sd_v, [sd_base + pos])
                found = (v == t) & valid
                plsc.store_scatter(present_v, [pos], gvec, mask=found)
                acc = acc | jnp.where(valid & (v < t), 1, 0).astype(jnp.int32)
                return acc | jnp.where(found, 2, 0).astype(jnp.int32)

            acc_m = jnp.max(probe_fast)
            need_slow = (acc_m & 1) > 0

            def probe_slow(jg, _):
                jbase = jg * 16
                valid = (jbase + lanes) < L
                t = hist_s[gi, pl.ds(jbase, 16)] ^ cur
                r = lax.shift_right_logical(t, BSH) & (NBKT - 1)
                pos = plsc.load_gather(off_v, [off_base + r])
                v = plsc.load_gather(sd_v, [sd_base + pos])

                def scan_cond(c):
                    _, vv = c
                    return jnp.any(vv < t)

                def scan_step(c):
                    pp, vv = c
                    pp = pp + jnp.where(vv < t, 1, 0).astype(jnp.int32)
                    return pp, plsc.load_gather(sd_v, [sd_base + pp])

                pos, v = lax.while_loop(scan_cond, scan_step, (pos, v))
                found = (v == t) & valid
                plsc.store_scatter(present_v, [pos], gvec, mask=found)
                return 0

            @pl.when(need_slow)
            def _():
                lax.fori_loop(0, ngrp, probe_slow, 0)

            game_hit = (acc_m >= 2) | need_slow
            rk0 = player * PADN

            @pl.when(game_hit)
            def _():
                for pg in range(NGRP):
                    r = rank_v[pl.ds(rk0 + pg * 16, 16)]
                    rep = plsc.load_gather(present_v, [r]) == gvec
                    lg = legal_s[gi, pl.ds(pg * 16, 16)]
                    out_buf[gi, pl.ds(pg * 16, 16)] = jnp.where(
                        (lg > 0) & rep, jnp.float32(0), lg)

            @pl.when(jnp.logical_not(game_hit))
            def _():
                for pg in range(NGRP):
                    out_buf[gi, pl.ds(pg * 16, 16)] = legal_s[
                        gi, pl.ds(pg * 16, 16)]
            return 0

        lax.fori_loop(0, CH, game_body, 0)

    def pair_body(cpair, _):
        for s in range(2):
            cc = cpair * 2 + s
            hist_s, legal_s, sem = bufs[s]
            for cp in in_copies(cc, hist_s, legal_s, sem):
                cp.wait()
            chunk_compute(cc, hist_s, legal_s)

            @pl.when(cc + 2 < NCHUNK)
            def _():
                for cp in in_copies(cc + 2, hist_s, legal_s, sem):
                    cp.start()

            pltpu.sync_copy(out_buf, out_hbm.at[pl.ds(base + cc * CH, CH)])
        return 0

    lax.fori_loop(0, NCHUNK // 2, pair_body, 0)


def kernel(legal_mask, Zpos, current_player, current_hash, hash_history,
           move_count):
    b, h, w = legal_mask.shape
    d = Zpos[:, 0][None, :] ^ jnp.stack([Zpos[:, 1], Zpos[:, 2]])
    sd = jnp.sort(d, axis=1)
    sd_pad = jnp.concatenate(
        [sd, jnp.full((2, TBL - N2), I32MAX, jnp.int32)], axis=1)
    rank = jnp.sum(d[:, None, :] < d[:, :, None], axis=2,
                   dtype=jnp.int32)
    rank_pad = jnp.concatenate(
        [rank, jnp.zeros((2, PADN - N2), jnp.int32)], axis=1)
    buck = lax.shift_right_logical(d, BSH)
    off = jnp.sum(
        buck[:, None, :] < jnp.arange(NBKT, dtype=jnp.int32)[None, :, None],
        axis=2, dtype=jnp.int32)

    hist_pad = jnp.pad(hash_history, ((0, 0), (0, PADN - N2)))
    legal_pad = jnp.pad(legal_mask.reshape(b, N2), ((0, 0), (0, PADN - N2)))

    mesh = plsc.VectorSubcoreMesh(core_axis_name="c", subcore_axis_name="s")
    kfn = pl.kernel(
        _sc_body,
        mesh=mesh,
        out_type=jax.ShapeDtypeStruct((B, PADN), jnp.float32),
        compiler_params=pltpu.CompilerParams(needs_layout_passes=False),
        scratch_types=[
            pltpu.VMEM((2 * TBL,), jnp.int32),
            pltpu.VMEM((2 * PADN,), jnp.int32),
            pltpu.VMEM((2 * NBKT,), jnp.int32),
            pltpu.VMEM((GPT + 16,), jnp.int32),
            pltpu.VMEM((GPT + 16,), jnp.int32),
            pltpu.VMEM((GPT + 16,), jnp.int32),
            pltpu.VMEM((TBL,), jnp.int32),
            pltpu.VMEM((CH, PADN), jnp.int32),
            pltpu.VMEM((CH, PADN), jnp.int32),
            pltpu.VMEM((CH, PADN), jnp.float32),
            pltpu.VMEM((CH, PADN), jnp.float32),
            pltpu.VMEM((CH, PADN), jnp.float32),
            pltpu.SemaphoreType.DMA,
            pltpu.SemaphoreType.DMA,
        ],
    )
    out = kfn(hist_pad, legal_pad, sd_pad.reshape(-1), rank_pad.reshape(-1),
              off.reshape(-1), current_hash, current_player, move_count)
    return out[:, :N2].reshape(b, h, w)


# speedup vs baseline: 1.3927x; 1.0562x over previous
import jax
import jax.numpy as jnp
from jax import lax
from jax.experimental import pallas as pl
from jax.experimental.pallas import tpu as pltpu
from jax.experimental.pallas import tpu_sc as plsc

B = 16384
N2 = 361
PADN = 368
NGRP = PADN // 16
TBL = 512
NBKT = 8192
BSH = 18
NW = 32
GPT = B // NW
CH = 32
NCHUNK = GPT // CH
I32MAX = 2147483647


def _sc_body(hist_hbm, sd_hbm, rank_hbm, off_hbm, cur_hbm, pl_hbm,
             mc_hbm, out_hbm, sd_v, rank_v, off_v, cur_v, pl_v, mc_v,
             present_v, hist0, hist1, out_buf, sem0, sem1):
    wid = lax.axis_index("s") * 2 + lax.axis_index("c")
    base = wid * GPT

    pltpu.sync_copy(sd_hbm, sd_v)
    pltpu.sync_copy(rank_hbm, rank_v)
    pltpu.sync_copy(off_hbm, off_v)
    pltpu.sync_copy(cur_hbm.at[pl.ds(base, GPT)], cur_v.at[pl.ds(0, GPT)])
    pltpu.sync_copy(pl_hbm.at[pl.ds(base, GPT)], pl_v.at[pl.ds(0, GPT)])
    pltpu.sync_copy(mc_hbm.at[pl.ds(base, GPT)], mc_v.at[pl.ds(0, GPT)])

    lanes = lax.iota(jnp.int32, 16)
    for k in range(TBL // 16):
        present_v[pl.ds(k * 16, 16)] = jnp.full((16,), -1, jnp.int32)

    bufs = ((hist0, sem0), (hist1, sem1))

    def in_copies(cc, hist_s, sem):
        rows = pl.ds(base + cc * CH, CH)
        return (pltpu.make_async_copy(hist_hbm.at[rows], hist_s, sem),)

    for s in range(2):
        for cp in in_copies(s, *bufs[s]):
            cp.start()

    def chunk_compute(cc, hist_s):
        def game_body(gi, _):
            g = cc * CH + gi
            cur = cur_v[pl.ds(g, 16)][0]
            player = lax.bitwise_and(pl_v[pl.ds(g, 16)][0], 1)
            L = lax.min(mc_v[pl.ds(g, 16)][0], N2)
            gvec = jnp.full((16,), g, jnp.int32)
            sd_base = jnp.full((16,), player * TBL, jnp.int32)
            off_base = jnp.full((16,), player * NBKT, jnp.int32)
            ngrp = (L + 15) // 16
            zero16 = jnp.zeros((16,), jnp.int32)

            @plsc.parallel_loop(0, ngrp, unroll=4, carry=zero16)
            def probe_fast(jg, acc):
                jbase = jg * 16
                valid = (jbase + lanes) < L
                t = hist_s[gi, pl.ds(jbase, 16)] ^ cur
                r = lax.shift_right_logical(t, BSH) & (NBKT - 1)
                pos = plsc.load_gather(off_v, [off_base + r])
                v = plsc.load_gather(sd_v, [sd_base + pos])
                pos = pos + jnp.where(v < t, 1, 0).astype(jnp.int32)
                v = plsc.load_gather(sd_v, [sd_base + pos])
                pos = pos + jnp.where(v < t, 1, 0).astype(jnp.int32)
                v = plsc.load_gather(sd_v, [sd_base + pos])
                found = (v == t) & valid
                plsc.store_scatter(present_v, [pos], gvec, mask=found)
                acc = acc | jnp.where(valid & (v < t), 1, 0).astype(jnp.int32)
                return acc | jnp.where(found, 2, 0).astype(jnp.int32)

            acc_m = jnp.max(probe_fast)
            need_slow = (acc_m & 1) > 0

            def probe_slow(jg, _):
                jbase = jg * 16
                valid = (jbase + lanes) < L
                t = hist_s[gi, pl.ds(jbase, 16)] ^ cur
                r = lax.shift_right_logical(t, BSH) & (NBKT - 1)
                pos = plsc.load_gather(off_v, [off_base + r])
                v = plsc.load_gather(sd_v, [sd_base + pos])

                def scan_cond(c):
                    _, vv = c
                    return jnp.any(vv < t)

                def scan_step(c):
                    pp, vv = c
                    pp = pp + jnp.where(vv < t, 1, 0).astype(jnp.int32)
                    return pp, plsc.load_gather(sd_v, [sd_base + pp])

                pos, v = lax.while_loop(scan_cond, scan_step, (pos, v))
                found = (v == t) & valid
                plsc.store_scatter(present_v, [pos], gvec, mask=found)
                return 0

            @pl.when(need_slow)
            def _():
                lax.fori_loop(0, ngrp, probe_slow, 0)

            game_hit = (acc_m >= 2) | need_slow
            rk0 = player * PADN
            zf = jnp.zeros((16,), jnp.float32)

            @pl.when(game_hit)
            def _():
                for pg in range(NGRP):
                    r = rank_v[pl.ds(rk0 + pg * 16, 16)]
                    rep = plsc.load_gather(present_v, [r]) == gvec
                    out_buf[gi, pl.ds(pg * 16, 16)] = jnp.where(
                        rep, jnp.float32(1), jnp.float32(0))

            @pl.when(jnp.logical_not(game_hit))
            def _():
                for pg in range(NGRP):
                    out_buf[gi, pl.ds(pg * 16, 16)] = zf
            return 0

        lax.fori_loop(0, CH, game_body, 0)

    def pair_body(cpair, _):
        for s in range(2):
            cc = cpair * 2 + s
            hist_s, sem = bufs[s]
            for cp in in_copies(cc, hist_s, sem):
                cp.wait()
            chunk_compute(cc, hist_s)

            @pl.when(cc + 2 < NCHUNK)
            def _():
                for cp in in_copies(cc + 2, hist_s, sem):
                    cp.start()

            pltpu.sync_copy(out_buf, out_hbm.at[pl.ds(base + cc * CH, CH)])
        return 0

    lax.fori_loop(0, NCHUNK // 2, pair_body, 0)


def kernel(legal_mask, Zpos, current_player, current_hash, hash_history,
           move_count):
    b, h, w = legal_mask.shape
    d = Zpos[:, 0][None, :] ^ jnp.stack([Zpos[:, 1], Zpos[:, 2]])
    sd = jnp.sort(d, axis=1)
    sd_pad = jnp.concatenate(
        [sd, jnp.full((2, TBL - N2), I32MAX, jnp.int32)], axis=1)
    rank = jnp.sum(d[:, None, :] < d[:, :, None], axis=2,
                   dtype=jnp.int32)
    rank_pad = jnp.concatenate(
        [rank, jnp.zeros((2, PADN - N2), jnp.int32)], axis=1)
    buck = lax.shift_right_logical(d, BSH)
    off = jnp.sum(
        buck[:, None, :] < jnp.arange(NBKT, dtype=jnp.int32)[None, :, None],
        axis=2, dtype=jnp.int32)

    hist_pad = jnp.pad(hash_history, ((0, 0), (0, PADN - N2)))

    mesh = plsc.VectorSubcoreMesh(core_axis_name="c", subcore_axis_name="s")
    kfn = pl.kernel(
        _sc_body,
        mesh=mesh,
        out_type=jax.ShapeDtypeStruct((B, PADN), jnp.float32),
        compiler_params=pltpu.CompilerParams(needs_layout_passes=False),
        scratch_types=[
            pltpu.VMEM((2 * TBL,), jnp.int32),
            pltpu.VMEM((2 * PADN,), jnp.int32),
            pltpu.VMEM((2 * NBKT,), jnp.int32),
            pltpu.VMEM((GPT + 16,), jnp.int32),
            pltpu.VMEM((GPT + 16,), jnp.int32),
            pltpu.VMEM((GPT + 16,), jnp.int32),
            pltpu.VMEM((TBL,), jnp.int32),
            pltpu.VMEM((CH, PADN), jnp.int32),
            pltpu.VMEM((CH, PADN), jnp.int32),
            pltpu.VMEM((CH, PADN), jnp.float32),
            pltpu.SemaphoreType.DMA,
            pltpu.SemaphoreType.DMA,
        ],
    )
    mask = kfn(hist_pad, sd_pad.reshape(-1), rank_pad.reshape(-1),
               off.reshape(-1), current_hash, current_player, move_count)
    return legal_mask * (1.0 - mask[:, :N2].reshape(b, h, w))
